# baseline (speedup 1.0000x reference)
import functools

import jax
import jax.numpy as jnp
from jax import lax
from jax.experimental import pallas as pl
from jax.experimental.pallas import tpu as pltpu
from jax.experimental.pallas import tpu_sc as plsc

N_NODES = 10000
N_EDGES = 320000
D = 128
NUM_RBF = 16
RBF_GAMMA = 10.0

NW = 32
NSEC = 5
NCHUNK = 25
CHUNK = 80
ACC_ROWS = 10240
ROWS_PER_TILE = ACC_ROWS // 16


def _rbf_all(r_floats, p_floats):
    NC3 = 250
    CB = 50

    def body(x_ref, p_ref, cb_ref, o_ref):
        x2 = x_ref[0].reshape(CB * 10, 8)
        xsel = jnp.dot(x2, p_ref[...], preferred_element_type=jnp.float32)
        d = xsel - cb_ref[...]
        v = jnp.exp(-RBF_GAMMA * d * d).reshape(CB, 10, D)
        o_ref[0] = jnp.concatenate(
            [v, jnp.zeros((CB, 6, D), jnp.float32)], axis=1
        )

    xin = jnp.concatenate(
        [r_floats.reshape(NW, NC3, 10, 4), p_floats.reshape(NW, NC3, 10, 4)],
        axis=3,
    )
    c = jnp.arange(D, dtype=jnp.int32)
    P = ((c >> 5) + 4 * ((c >> 4) & 1) == jnp.arange(8)[:, None]).astype(
        jnp.float32
    )
    cb = ((c & 15).astype(jnp.float32) / (NUM_RBF - 1.0)).reshape(1, D)

    return pl.pallas_call(
        body,
        grid=(NW, NC3 // CB),
        in_specs=[
            pl.BlockSpec((1, CB, 10, 8), lambda i, j: (i, j, 0, 0)),
            pl.BlockSpec((8, D), lambda i, j: (0, 0)),
            pl.BlockSpec((1, D), lambda i, j: (0, 0)),
        ],
        out_specs=pl.BlockSpec((1, CB, 16, D), lambda i, j: (i, j, 0, 0)),
        out_shape=jax.ShapeDtypeStruct((NW, NC3, 16, D), jnp.float32),
    )(xin, P, cb)


def _sc_gather_scatter(node_repr, src3, dst3):
    mesh = plsc.VectorSubcoreMesh(core_axis_name="c", subcore_axis_name="s")

    @functools.partial(
        pl.kernel,
        mesh=mesh,
        out_type=jax.ShapeDtypeStruct((2, ACC_ROWS, D), jnp.float32),
        scratch_types=[
            pltpu.VMEM((NCHUNK, CHUNK), jnp.int32),
            pltpu.VMEM((NCHUNK, CHUNK), jnp.int32),
            pltpu.VMEM((CHUNK, D), jnp.float32),
            pltpu.VMEM((CHUNK, D), jnp.float32),
            pltpu.VMEM_SHARED((ACC_ROWS, D), jnp.float32),
            pltpu.SemaphoreType.DMA,
            pltpu.SemaphoreType.DMA,
        ],
    )
    def k(a_hbm, src_hbm, dst_hbm, out_hbm,
          src_v, dst_v, rows0, rows1, acc, sem0, sem1):
        cid = lax.axis_index("c")
        sid = lax.axis_index("s")
        wid = sid * 2 + cid

        def zloop(i, _):
            rows0[i // 8, pl.ds((i % 8) * 16, 16)] = jnp.zeros((16,), jnp.float32)
            return _
        lax.fori_loop(0, CHUNK * (D // 16), zloop, None)
        for t in range(ROWS_PER_TILE // CHUNK):
            pltpu.sync_copy(
                rows0, acc.at[pl.ds(sid * ROWS_PER_TILE + t * CHUNK, CHUNK)]
            )
        plsc.subcore_barrier()

        bufs = (rows0, rows1)
        sems = (sem0, sem1)
        for s in range(NSEC):
            pltpu.sync_copy(src_hbm.at[wid, s], src_v)
            pltpu.sync_copy(dst_hbm.at[wid, s], dst_v)
            d0 = pltpu.async_copy(a_hbm.at[src_v.at[0]], rows0, sem0)

            def chunk(j, _):
                @pl.when(j + 1 < NCHUNK)
                def _issue():
                    for b in range(2):
                        @pl.when(lax.rem(j + 1, 2) == b)
                        def _():
                            pltpu.async_copy(
                                a_hbm.at[src_v.at[j + 1]], bufs[b], sems[b]
                            )
                for b in range(2):
                    @pl.when(lax.rem(j, 2) == b)
                    def _():
                        pltpu.make_async_copy(
                            a_hbm.at[src_v.at[j]], bufs[b], sems[b]
                        ).wait()
                        pltpu.sync_copy(bufs[b], acc.at[dst_v.at[j]], add=True)
                return _
            lax.fori_loop(0, NCHUNK, chunk, None)

        plsc.subcore_barrier()
        pltpu.sync_copy(
            acc.at[pl.ds(sid * ROWS_PER_TILE, ROWS_PER_TILE)],
            out_hbm.at[cid, pl.ds(sid * ROWS_PER_TILE, ROWS_PER_TILE)],
        )

    return k(node_repr, src3, dst3)


CH3 = 40
NCH3 = 250
SEC3 = 5
CPS3 = NCH3 // SEC3


def _sc_aux(labs4, rbf4, dst3):
    mesh = plsc.VectorSubcoreMesh(core_axis_name="c", subcore_axis_name="s")

    @functools.partial(
        pl.kernel,
        mesh=mesh,
        out_type=jax.ShapeDtypeStruct((2, ACC_ROWS, D), jnp.float32),
        scratch_types=[
            pltpu.VMEM((CPS3, CH3), jnp.int32),
            pltpu.VMEM((CH3 // 2, 16), jnp.int32),
            pltpu.VMEM((CH3 // 2, 16), jnp.int32),
            pltpu.VMEM((16, D), jnp.float32),
            pltpu.VMEM((16, D), jnp.float32),
            pltpu.VMEM((CH3, D), jnp.float32),
            pltpu.VMEM((CH3, D), jnp.float32),
            pltpu.VMEM_SHARED((ACC_ROWS, D), jnp.float32),
            pltpu.SemaphoreType.DMA,
            pltpu.SemaphoreType.DMA,
            pltpu.SemaphoreType.DMA,
            pltpu.SemaphoreType.DMA,
            pltpu.SemaphoreType.DMA,
            pltpu.SemaphoreType.DMA,
        ],
    )
    def k(labs_hbm, rbf_hbm, dst_hbm, out_hbm,
          dst_v, labs0, labs1, rbf0, rbf1, aux0, aux1, acc,
          semL0, semL1, semR0, semR1, semS0, semS1):
        cid = lax.axis_index("c")
        sid = lax.axis_index("s")
        wid = sid * 2 + cid
        labs_b = (labs0, labs1)
        rbf_b = (rbf0, rbf1)
        aux_b = (aux0, aux1)
        semL = (semL0, semL1)
        semR = (semR0, semR1)
        semS = (semS0, semS1)

        def zloop(i, _):
            z = jnp.zeros((16,), jnp.float32)
            aux0[i // 8, pl.ds((i % 8) * 16, 16)] = z
            aux1[i // 8, pl.ds((i % 8) * 16, 16)] = z
            return _
        lax.fori_loop(0, CH3 * (D // 16), zloop, None)
        for t in range(ROWS_PER_TILE // CH3):
            pltpu.sync_copy(
                aux0, acc.at[pl.ds(sid * ROWS_PER_TILE + t * CH3, CH3)]
            )
        plsc.subcore_barrier()

        for s in range(SEC3):
            pltpu.sync_copy(dst_hbm.at[wid, s], dst_v)
            pltpu.async_copy(labs_hbm.at[wid, s * CPS3], labs0, semL0)
            pltpu.async_copy(rbf_hbm.at[wid, s * CPS3], rbf0, semR0)

            def chunk(j, _):
                J = s * CPS3 + j
                for b in range(2):
                    @pl.when(lax.rem(j, 2) == b)
                    def _body():
                        pltpu.make_async_copy(
                            labs_hbm.at[wid, J], labs_b[b], semL[b]
                        ).wait()
                        pltpu.make_async_copy(
                            rbf_hbm.at[wid, J], rbf_b[b], semR[b]
                        ).wait()

                        @pl.when(j + 1 < CPS3)
                        def _prefetch():
                            pltpu.async_copy(
                                labs_hbm.at[wid, J + 1], labs_b[1 - b],
                                semL[1 - b]
                            )
                            pltpu.async_copy(
                                rbf_hbm.at[wid, J + 1], rbf_b[1 - b],
                                semR[1 - b]
                            )

                        @pl.when(j >= 2)
                        def _drain():
                            pltpu.make_async_copy(
                                aux_b[b], acc.at[dst_v.at[j]], semS[b]
                            ).wait()

                        def rloop(rp, _3):
                            io = lax.iota(jnp.int32, 16)
                            one = jnp.ones((16,), jnp.float32)
                            zero = jnp.zeros((16,), jnp.float32)
                            lv = labs_b[b][rp]
                            q = rp // 2
                            m = (rp % 2) * 64
                            for h in range(2):
                                r = 2 * rp + h
                                o = 8 * h
                                aux_b[b][r, pl.ds(0, 16)] = jnp.where(
                                    io == lv[o + 0], one, zero
                                ) + jnp.where(io == lv[o + 1] + 8, one, zero)
                                aux_b[b][r, pl.ds(16, 16)] = jnp.where(
                                    io == lv[o + 2], one, zero
                                ) + jnp.where(io == lv[o + 3] + 8, one, zero)
                                aux_b[b][r, pl.ds(32, 16)] = jnp.where(
                                    io == lv[o + 4], one, zero
                                ) + jnp.where(io == lv[o + 5] + 8, one, zero)
                                aux_b[b][r, pl.ds(48, 16)] = rbf_b[b][
                                    q, pl.ds(m + 32 * h, 16)
                                ]
                                aux_b[b][r, pl.ds(64, 16)] = rbf_b[b][
                                    q, pl.ds(m + 32 * h + 16, 16)
                                ]
                            return _3
                        lax.fori_loop(0, CH3 // 2, rloop, None)

                        pltpu.async_copy(
                            aux_b[b], acc.at[dst_v.at[j]], semS[b], add=True
                        )
                return _
            lax.fori_loop(0, CPS3, chunk, None)

            pltpu.make_async_copy(
                aux0, acc.at[dst_v.at[CPS3 - 2]], semS0
            ).wait()
            pltpu.make_async_copy(
                aux1, acc.at[dst_v.at[CPS3 - 1]], semS1
            ).wait()

        plsc.subcore_barrier()
        pltpu.sync_copy(
            acc.at[pl.ds(sid * ROWS_PER_TILE, ROWS_PER_TILE)],
            out_hbm.at[cid, pl.ds(sid * ROWS_PER_TILE, ROWS_PER_TILE)],
        )

    return k(labs4, rbf4, dst3)


def _mlp(pA, pX, Waux, W1, b1, W2, b2):
    NB = 2048
    NROWS = pA.shape[1]

    def body(pa_ref, px_ref, wa, w1, b1r, w2, b2r, o_ref):
        agg = pa_ref[0] + pa_ref[1]
        aux = px_ref[0] + px_ref[1]
        agg = agg + jnp.dot(aux, wa[...], preferred_element_type=jnp.float32)
        h = jnp.maximum(
            jnp.dot(agg, w1[...], preferred_element_type=jnp.float32) + b1r[...], 0.0
        )
        o_ref[...] = jnp.maximum(
            jnp.dot(h, w2[...], preferred_element_type=jnp.float32) + b2r[...], 0.0
        )

    return pl.pallas_call(
        body,
        grid=(NROWS // NB,),
        in_specs=[
            pl.BlockSpec((2, NB, D), lambda i: (0, i, 0)),
            pl.BlockSpec((2, NB, D), lambda i: (0, i, 0)),
            pl.BlockSpec((D, D), lambda i: (0, 0)),
            pl.BlockSpec((D, 2 * D), lambda i: (0, 0)),
            pl.BlockSpec((1, 2 * D), lambda i: (0, 0)),
            pl.BlockSpec((2 * D, D), lambda i: (0, 0)),
            pl.BlockSpec((1, D), lambda i: (0, 0)),
        ],
        out_specs=pl.BlockSpec((NB, D), lambda i: (i, 0)),
        out_shape=jax.ShapeDtypeStruct((NROWS, D), jnp.float32),
    )(pA, pX, Waux, W1, b1, W2, b2)


def kernel(superimposed_atom_repr, edge_index, r_labels, p_labels, r_floats,
           p_floats, emb0, emb1, emb2, W_rbf, b_rbf, W1, b1, W2, b2):
    Wc = jnp.concatenate([emb0, emb1, emb2], axis=0)
    z64 = jnp.zeros((64,), jnp.float32)
    bias_row = jnp.concatenate([b_rbf, z64]).reshape(1, D)
    r_rows = jnp.concatenate([Wc, -Wc], 1)
    r_rows = r_rows.at[0:8].add(bias_row)
    p_rows = jnp.concatenate([jnp.zeros_like(Wc), Wc], 1)
    rbf_r_rows = jnp.concatenate([W_rbf, -W_rbf], 1)
    rbf_p_rows = jnp.concatenate([jnp.zeros_like(W_rbf), W_rbf], 1)
    Waux = jnp.concatenate(
        [r_rows, p_rows, rbf_r_rows, rbf_p_rows,
         jnp.zeros((48, D), jnp.float32)], 0
    )

    src3 = edge_index[0].reshape(NW, NSEC, NCHUNK, CHUNK)
    dst3 = edge_index[1].reshape(NW, NSEC, NCHUNK, CHUNK)
    dstK3 = edge_index[1].reshape(NW, SEC3, CPS3, CH3)
    labs8 = jnp.concatenate(
        [r_labels, p_labels, jnp.zeros((N_EDGES, 2), jnp.int32)], axis=1
    ).reshape(NW, NCH3, CH3 // 2, 16)

    rbf4 = _rbf_all(r_floats, p_floats)

    pA = _sc_gather_scatter(superimposed_atom_repr, src3, dst3)
    pX = _sc_aux(labs8, rbf4, dstK3)

    out = _mlp(pA, pX, Waux, W1, b1.reshape(1, 2 * D), W2, b2.reshape(1, D))
    return out[:N_NODES]


# speedup vs baseline: 1.2735x; 1.2735x over previous
import functools

import jax
import jax.numpy as jnp
from jax import lax
from jax.experimental import pallas as pl
from jax.experimental.pallas import tpu as pltpu
from jax.experimental.pallas import tpu_sc as plsc

N_NODES = 10000
N_EDGES = 320000
D = 128
NUM_RBF = 16
RBF_GAMMA = 10.0

NW = 32
NSEC = 5
NCHUNK = 25
CHUNK = 80
ACC_ROWS = 10240
ROWS_PER_TILE = ACC_ROWS // 16


def _rbf_all(r_floats, p_floats):
    NC3 = 250
    CB = 250

    def body(xr_ref, xp_ref, o_ref):
        xr = xr_ref[0]
        xp = xp_ref[0]
        c = lax.broadcasted_iota(jnp.int32, (CB, 40, 2 * NUM_RBF), 2)
        center = (c % NUM_RBF).astype(jnp.float32) / (NUM_RBF - 1.0)
        x = jnp.where(c < NUM_RBF, xr[:, :, None], xp[:, :, None])
        d = x - center
        o_ref[0] = jnp.exp(-RBF_GAMMA * d * d)

    return pl.pallas_call(
        body,
        grid=(NW,),
        in_specs=[
            pl.BlockSpec((1, CB, 40), lambda i: (i, 0, 0)),
            pl.BlockSpec((1, CB, 40), lambda i: (i, 0, 0)),
        ],
        out_specs=pl.BlockSpec(
            (1, CB, 40, 2 * NUM_RBF), lambda i: (i, 0, 0, 0)
        ),
        out_shape=jax.ShapeDtypeStruct(
            (NW, NC3, 40, 2 * NUM_RBF), jnp.float32
        ),
    )(
        r_floats.reshape(NW, NC3, 40),
        p_floats.reshape(NW, NC3, 40),
    )


def _sc_gather_scatter(node_repr, src3, dst3):
    mesh = plsc.VectorSubcoreMesh(core_axis_name="c", subcore_axis_name="s")

    @functools.partial(
        pl.kernel,
        mesh=mesh,
        out_type=jax.ShapeDtypeStruct((2, ACC_ROWS, D), jnp.float32),
        scratch_types=[
            pltpu.VMEM((NCHUNK, CHUNK), jnp.int32),
            pltpu.VMEM((NCHUNK, CHUNK), jnp.int32),
            pltpu.VMEM((CHUNK, D), jnp.float32),
            pltpu.VMEM((CHUNK, D), jnp.float32),
            pltpu.VMEM_SHARED((ACC_ROWS, D), jnp.float32),
            pltpu.SemaphoreType.DMA,
            pltpu.SemaphoreType.DMA,
        ],
    )
    def k(a_hbm, src_hbm, dst_hbm, out_hbm,
          src_v, dst_v, rows0, rows1, acc, sem0, sem1):
        cid = lax.axis_index("c")
        sid = lax.axis_index("s")
        wid = sid * 2 + cid

        def zloop(i, _):
            rows0[i // 8, pl.ds((i % 8) * 16, 16)] = jnp.zeros((16,), jnp.float32)
            return _
        lax.fori_loop(0, CHUNK * (D // 16), zloop, None)
        for t in range(ROWS_PER_TILE // CHUNK):
            pltpu.sync_copy(
                rows0, acc.at[pl.ds(sid * ROWS_PER_TILE + t * CHUNK, CHUNK)]
            )
        plsc.subcore_barrier()

        bufs = (rows0, rows1)
        sems = (sem0, sem1)
        for s in range(NSEC):
            pltpu.sync_copy(src_hbm.at[wid, s], src_v)
            pltpu.sync_copy(dst_hbm.at[wid, s], dst_v)
            d0 = pltpu.async_copy(a_hbm.at[src_v.at[0]], rows0, sem0)

            def chunk(j, _):
                @pl.when(j + 1 < NCHUNK)
                def _issue():
                    for b in range(2):
                        @pl.when(lax.rem(j + 1, 2) == b)
                        def _():
                            pltpu.async_copy(
                                a_hbm.at[src_v.at[j + 1]], bufs[b], sems[b]
                            )
                for b in range(2):
                    @pl.when(lax.rem(j, 2) == b)
                    def _():
                        pltpu.make_async_copy(
                            a_hbm.at[src_v.at[j]], bufs[b], sems[b]
                        ).wait()
                        pltpu.sync_copy(bufs[b], acc.at[dst_v.at[j]], add=True)
                return _
            lax.fori_loop(0, NCHUNK, chunk, None)

        plsc.subcore_barrier()
        pltpu.sync_copy(
            acc.at[pl.ds(sid * ROWS_PER_TILE, ROWS_PER_TILE)],
            out_hbm.at[cid, pl.ds(sid * ROWS_PER_TILE, ROWS_PER_TILE)],
        )

    return k(node_repr, src3, dst3)


CH3 = 40
NCH3 = 250
SEC3 = 5
CPS3 = NCH3 // SEC3


def _sc_aux(labs4, rbf4, dst3):
    mesh = plsc.VectorSubcoreMesh(core_axis_name="c", subcore_axis_name="s")

    @functools.partial(
        pl.kernel,
        mesh=mesh,
        out_type=jax.ShapeDtypeStruct((2, ACC_ROWS, D), jnp.float32),
        scratch_types=[
            pltpu.VMEM((CPS3, CH3), jnp.int32),
            pltpu.VMEM((CH3 // 2, 16), jnp.int32),
            pltpu.VMEM((CH3 // 2, 16), jnp.int32),
            pltpu.VMEM((CH3, 2 * NUM_RBF), jnp.float32),
            pltpu.VMEM((CH3, 2 * NUM_RBF), jnp.float32),
            pltpu.VMEM((CH3, D), jnp.float32),
            pltpu.VMEM((CH3, D), jnp.float32),
            pltpu.VMEM_SHARED((ACC_ROWS, D), jnp.float32),
            pltpu.SemaphoreType.DMA,
            pltpu.SemaphoreType.DMA,
            pltpu.SemaphoreType.DMA,
            pltpu.SemaphoreType.DMA,
            pltpu.SemaphoreType.DMA,
            pltpu.SemaphoreType.DMA,
        ],
    )
    def k(labs_hbm, rbf_hbm, dst_hbm, out_hbm,
          dst_v, labs0, labs1, rbf0, rbf1, aux0, aux1, acc,
          semL0, semL1, semR0, semR1, semS0, semS1):
        cid = lax.axis_index("c")
        sid = lax.axis_index("s")
        wid = sid * 2 + cid
        labs_b = (labs0, labs1)
        rbf_b = (rbf0, rbf1)
        aux_b = (aux0, aux1)
        semL = (semL0, semL1)
        semR = (semR0, semR1)
        semS = (semS0, semS1)

        def zloop(i, _):
            z = jnp.zeros((16,), jnp.float32)
            aux0[i // 8, pl.ds((i % 8) * 16, 16)] = z
            aux1[i // 8, pl.ds((i % 8) * 16, 16)] = z
            return _
        lax.fori_loop(0, CH3 * (D // 16), zloop, None)
        for t in range(ROWS_PER_TILE // CH3):
            pltpu.sync_copy(
                aux0, acc.at[pl.ds(sid * ROWS_PER_TILE + t * CH3, CH3)]
            )
        plsc.subcore_barrier()

        for s in range(SEC3):
            pltpu.sync_copy(dst_hbm.at[wid, s], dst_v)
            pltpu.async_copy(labs_hbm.at[wid, s * CPS3], labs0, semL0)
            pltpu.async_copy(rbf_hbm.at[wid, s * CPS3], rbf0, semR0)

            def chunk(j, _):
                J = s * CPS3 + j
                for b in range(2):
                    @pl.when(lax.rem(j, 2) == b)
                    def _body():
                        pltpu.make_async_copy(
                            labs_hbm.at[wid, J], labs_b[b], semL[b]
                        ).wait()
                        pltpu.make_async_copy(
                            rbf_hbm.at[wid, J], rbf_b[b], semR[b]
                        ).wait()

                        @pl.when(j + 1 < CPS3)
                        def _prefetch():
                            pltpu.async_copy(
                                labs_hbm.at[wid, J + 1], labs_b[1 - b],
                                semL[1 - b]
                            )
                            pltpu.async_copy(
                                rbf_hbm.at[wid, J + 1], rbf_b[1 - b],
                                semR[1 - b]
                            )

                        @pl.when(j >= 2)
                        def _drain():
                            pltpu.make_async_copy(
                                aux_b[b], acc.at[dst_v.at[j]], semS[b]
                            ).wait()

                        def rloop(rp, _3):
                            io = lax.iota(jnp.int32, 16)
                            one = jnp.ones((16,), jnp.float32)
                            zero = jnp.zeros((16,), jnp.float32)
                            lv = labs_b[b][rp]
                            for h in range(2):
                                r = 2 * rp + h
                                o = 8 * h
                                aux_b[b][r, pl.ds(0, 16)] = jnp.where(
                                    io == lv[o + 0], one, zero
                                ) + jnp.where(io == lv[o + 1] + 8, one, zero)
                                aux_b[b][r, pl.ds(16, 16)] = jnp.where(
                                    io == lv[o + 2], one, zero
                                ) + jnp.where(io == lv[o + 3] + 8, one, zero)
                                aux_b[b][r, pl.ds(32, 16)] = jnp.where(
                                    io == lv[o + 4], one, zero
                                ) + jnp.where(io == lv[o + 5] + 8, one, zero)
                                aux_b[b][r, pl.ds(48, 16)] = rbf_b[b][
                                    r, pl.ds(0, 16)
                                ]
                                aux_b[b][r, pl.ds(64, 16)] = rbf_b[b][
                                    r, pl.ds(16, 16)
                                ]
                            return _3
                        lax.fori_loop(0, CH3 // 2, rloop, None)

                        pltpu.async_copy(
                            aux_b[b], acc.at[dst_v.at[j]], semS[b], add=True
                        )
                return _
            lax.fori_loop(0, CPS3, chunk, None)

            pltpu.make_async_copy(
                aux0, acc.at[dst_v.at[CPS3 - 2]], semS0
            ).wait()
            pltpu.make_async_copy(
                aux1, acc.at[dst_v.at[CPS3 - 1]], semS1
            ).wait()

        plsc.subcore_barrier()
        pltpu.sync_copy(
            acc.at[pl.ds(sid * ROWS_PER_TILE, ROWS_PER_TILE)],
            out_hbm.at[cid, pl.ds(sid * ROWS_PER_TILE, ROWS_PER_TILE)],
        )

    return k(labs4, rbf4, dst3)


def _mlp(pA, pX, Waux, W1, b1, W2, b2):
    NB = 2048
    NROWS = pA.shape[1]

    def body(pa_ref, px_ref, wa, w1, b1r, w2, b2r, o_ref):
        agg = pa_ref[0] + pa_ref[1]
        aux = px_ref[0] + px_ref[1]
        agg = agg + jnp.dot(aux, wa[...], preferred_element_type=jnp.float32)
        h = jnp.maximum(
            jnp.dot(agg, w1[...], preferred_element_type=jnp.float32) + b1r[...], 0.0
        )
        o_ref[...] = jnp.maximum(
            jnp.dot(h, w2[...], preferred_element_type=jnp.float32) + b2r[...], 0.0
        )

    return pl.pallas_call(
        body,
        grid=(NROWS // NB,),
        in_specs=[
            pl.BlockSpec((2, NB, D), lambda i: (0, i, 0)),
            pl.BlockSpec((2, NB, D), lambda i: (0, i, 0)),
            pl.BlockSpec((D, D), lambda i: (0, 0)),
            pl.BlockSpec((D, 2 * D), lambda i: (0, 0)),
            pl.BlockSpec((1, 2 * D), lambda i: (0, 0)),
            pl.BlockSpec((2 * D, D), lambda i: (0, 0)),
            pl.BlockSpec((1, D), lambda i: (0, 0)),
        ],
        out_specs=pl.BlockSpec((NB, D), lambda i: (i, 0)),
        out_shape=jax.ShapeDtypeStruct((NROWS, D), jnp.float32),
    )(pA, pX, Waux, W1, b1, W2, b2)


def kernel(superimposed_atom_repr, edge_index, r_labels, p_labels, r_floats,
           p_floats, emb0, emb1, emb2, W_rbf, b_rbf, W1, b1, W2, b2):
    Wc = jnp.concatenate([emb0, emb1, emb2], axis=0)
    z64 = jnp.zeros((64,), jnp.float32)
    bias_row = jnp.concatenate([b_rbf, z64]).reshape(1, D)
    r_rows = jnp.concatenate([Wc, -Wc], 1)
    r_rows = r_rows.at[0:8].add(bias_row)
    p_rows = jnp.concatenate([jnp.zeros_like(Wc), Wc], 1)
    rbf_r_rows = jnp.concatenate([W_rbf, -W_rbf], 1)
    rbf_p_rows = jnp.concatenate([jnp.zeros_like(W_rbf), W_rbf], 1)
    Waux = jnp.concatenate(
        [r_rows, p_rows, rbf_r_rows, rbf_p_rows,
         jnp.zeros((48, D), jnp.float32)], 0
    )

    src3 = edge_index[0].reshape(NW, NSEC, NCHUNK, CHUNK)
    dst3 = edge_index[1].reshape(NW, NSEC, NCHUNK, CHUNK)
    dstK3 = edge_index[1].reshape(NW, SEC3, CPS3, CH3)
    labs8 = jnp.concatenate(
        [r_labels, p_labels, jnp.zeros((N_EDGES, 2), jnp.int32)], axis=1
    ).reshape(NW, NCH3, CH3 // 2, 16)

    rbf4 = _rbf_all(r_floats, p_floats)

    pA = _sc_gather_scatter(superimposed_atom_repr, src3, dst3)
    pX = _sc_aux(labs8, rbf4, dstK3)

    out = _mlp(pA, pX, Waux, W1, b1.reshape(1, 2 * D), W2, b2.reshape(1, D))
    return out[:N_NODES]
